# Initial kernel scaffold; baseline (speedup 1.0000x reference)
import jax
import jax.numpy as jnp
from jax.experimental import pallas as pl


def kernel(x, W, b):
    raise NotImplementedError("write your pallas kernel here")


# speedup vs baseline: 23.6061x; 23.6061x over previous
import functools

import jax
import jax.numpy as jnp
from jax.experimental import pallas as pl
from jax.experimental.pallas import tpu as pltpu


_K = 16


def _edge_conv_body(xb_ref, xt_ref, w_ref, bias_ref, out_ref, *, n_rows, n_ch):
    xb = xb_ref[0]
    xt = xt_ref[0]
    w = w_ref[...]
    wc = w[:, :n_ch]
    wd = w[:, n_ch:]

    ss_col = jnp.sum(xb * xb, axis=0, keepdims=True)
    g_mat = jax.lax.dot_general(
        xt, xb, (((0,), (0,)), ((), ())),
        preferred_element_type=jnp.float32)
    dist = ss_col - 2.0 * g_mat

    u = jax.lax.dot_general(
        xt, wc - wd, (((0,), (1,)), ((), ())),
        preferred_element_type=jnp.float32)
    v_all = jax.lax.dot_general(
        xb, wd, (((0,), (1,)), ((), ())),
        preferred_element_type=jnp.float32)

    g = jnp.full((n_rows, v_all.shape[1]), -jnp.inf, dtype=jnp.float32)
    for _ in range(_K):
        m = jnp.min(dist, axis=1, keepdims=True)
        sel = (dist <= m)
        onehot = sel.astype(jnp.float32)
        picked = jax.lax.dot_general(
            onehot, v_all, (((1,), (0,)), ((), ())),
            preferred_element_type=jnp.float32)
        g = jnp.maximum(g, picked)
        dist = jnp.where(sel, jnp.inf, dist)

    out_ref[0] = jnp.maximum(u + g + bias_ref[...], 0.0)


@jax.jit
def kernel(x, W, b):
    batch, n_ch, n_nodes = x.shape
    n_out = W.shape[0]
    rows = 256
    grid = (batch, n_nodes // rows)

    out_nlast = pl.pallas_call(
        functools.partial(_edge_conv_body, n_rows=rows, n_ch=n_ch),
        grid=grid,
        in_specs=[
            pl.BlockSpec((1, n_ch, n_nodes), lambda bi, ri: (bi, 0, 0)),
            pl.BlockSpec((1, n_ch, rows), lambda bi, ri: (bi, 0, ri)),
            pl.BlockSpec((n_out, 2 * n_ch), lambda bi, ri: (0, 0)),
            pl.BlockSpec((1, n_out), lambda bi, ri: (0, 0)),
        ],
        out_specs=pl.BlockSpec((1, rows, n_out), lambda bi, ri: (bi, ri, 0)),
        out_shape=jax.ShapeDtypeStruct((batch, n_nodes, n_out), jnp.float32),
        compiler_params=pltpu.CompilerParams(
            dimension_semantics=("parallel", "arbitrary"),
        ),
    )(x, x, W, b.reshape(1, n_out))

    return jnp.transpose(out_nlast, (0, 2, 1))
